# baseline (speedup 1.0000x reference)
import functools

import jax
import jax.numpy as jnp
from jax import lax
from jax.experimental import pallas as pl
from jax.experimental.pallas import tpu as pltpu
from jax.experimental.pallas import tpu_sc as plsc

N = 10000
IN = 256
OUT = 256
H = 2
NEG = 0.2
E_RAW = 160000
E_TRUE = E_RAW + N
ROW = 128
EP = 184320
NROWS = EP // ROW
NC = 2
NS = 16
B_ROWS = NROWS // (NC * NS)
C_ROWS = NROWS // NS
NB = 10
BN_ = N // NB


def _proj_body(x_ref, w_ref, asrc_ref, adst_ref,
               h00, h01, h10, h11, as0, as1, ad0, ad1):
    xb = x_ref[...]
    h = jnp.dot(xb, w_ref[...], preferred_element_type=jnp.float32)
    h0 = h[:, :OUT]
    h1 = h[:, OUT:]
    h00[...] = h0[:, :128]
    h01[...] = h0[:, 128:]
    h10[...] = h1[:, :128]
    h11[...] = h1[:, 128:]
    asrc = asrc_ref[...]
    adst = adst_ref[...]
    as0[...] = jnp.sum(h0 * asrc[0][None, :], axis=-1, keepdims=True)
    as1[...] = jnp.sum(h1 * asrc[1][None, :], axis=-1, keepdims=True)
    ad0[...] = jnp.sum(h0 * adst[0][None, :], axis=-1, keepdims=True)
    ad1[...] = jnp.sum(h1 * adst[1][None, :], axis=-1, keepdims=True)


def _project(x, W, att_src, att_dst):
    hspec = pl.BlockSpec((BN_, 128), lambda i: (i, 0))
    aspec = pl.BlockSpec((BN_, 1), lambda i: (i, 0))
    return pl.pallas_call(
        _proj_body,
        grid=(NB,),
        in_specs=[
            pl.BlockSpec((BN_, IN), lambda i: (i, 0)),
            pl.BlockSpec((IN, H * OUT), lambda i: (0, 0)),
            pl.BlockSpec((H, OUT), lambda i: (0, 0)),
            pl.BlockSpec((H, OUT), lambda i: (0, 0)),
        ],
        out_specs=[hspec] * 4 + [aspec] * 4,
        out_shape=[jax.ShapeDtypeStruct((N, 128), jnp.float32)] * 4
        + [jax.ShapeDtypeStruct((N, 1), jnp.float32)] * 4,
    )(x, W, att_src, att_dst)


def _edge_weights_body(src_hbm, dst_hbm, as0_hbm, as1_hbm, ad0_hbm, ad1_hbm,
                       zn_hbm, ex0_hbm, ex1_hbm, denp_hbm,
                       as0_v, as1_v, ad0_v, ad1_v, srcc, dstc, ex0c, ex1c,
                       d0_sh, d1_sh):
    c = lax.axis_index("c")
    s = lax.axis_index("s")
    wid = s * NC + c
    pltpu.sync_copy(as0_hbm, as0_v)
    pltpu.sync_copy(as1_hbm, as1_v)
    pltpu.sync_copy(ad0_hbm, ad0_v)
    pltpu.sync_copy(ad1_hbm, ad1_v)

    @pl.when(s == 0)
    def _():
        pltpu.sync_copy(zn_hbm, d0_sh)
        pltpu.sync_copy(zn_hbm, d1_sh)

    plsc.subcore_barrier()

    base = wid * B_ROWS

    @pl.loop(0, B_ROWS)
    def _(j):
        r = base + j
        pltpu.sync_copy(src_hbm.at[r], srcc)
        pltpu.sync_copy(dst_hbm.at[r], dstc)
        for i in range(ROW // 16):
            s16 = srcc[pl.ds(i * 16, 16)]
            d16 = dstc[pl.ds(i * 16, 16)]
            gid = r * ROW + i * 16 + lax.iota(jnp.int32, 16)
            valid = gid < E_TRUE
            e0 = plsc.load_gather(as0_v, [s16]) + plsc.load_gather(ad0_v, [d16])
            e0 = jnp.where(e0 > 0, e0, NEG * e0)
            x0 = jnp.where(valid, jnp.exp(e0), 0.0)
            ex0c[pl.ds(i * 16, 16)] = x0
            e1 = plsc.load_gather(as1_v, [s16]) + plsc.load_gather(ad1_v, [d16])
            e1 = jnp.where(e1 > 0, e1, NEG * e1)
            x1 = jnp.where(valid, jnp.exp(e1), 0.0)
            ex1c[pl.ds(i * 16, 16)] = x1
        pltpu.sync_copy(ex0c, ex0_hbm.at[r])
        pltpu.sync_copy(ex1c, ex1_hbm.at[r])
        pltpu.sync_copy(ex0c, d0_sh.at[dstc], add=True)
        pltpu.sync_copy(ex1c, d1_sh.at[dstc], add=True)

    plsc.subcore_barrier()

    @pl.when(s == 0)
    def _():
        pltpu.sync_copy(d0_sh, denp_hbm.at[c, 0])
        pltpu.sync_copy(d1_sh, denp_hbm.at[c, 1])


def _edge_weights(src2d, dst2d, as0, as1, ad0, ad1, zn):
    mesh = plsc.VectorSubcoreMesh(core_axis_name="c", subcore_axis_name="s",
                                  num_cores=NC, num_subcores=NS)
    f = functools.partial(
        pl.kernel,
        out_type=[
            jax.ShapeDtypeStruct((NROWS, ROW), jnp.float32),
            jax.ShapeDtypeStruct((NROWS, ROW), jnp.float32),
            jax.ShapeDtypeStruct((NC, H, N), jnp.float32),
        ],
        mesh=mesh,
        compiler_params=pltpu.CompilerParams(needs_layout_passes=False),
        scratch_types=[
            pltpu.VMEM((N,), jnp.float32),
            pltpu.VMEM((N,), jnp.float32),
            pltpu.VMEM((N,), jnp.float32),
            pltpu.VMEM((N,), jnp.float32),
            pltpu.VMEM((ROW,), jnp.int32),
            pltpu.VMEM((ROW,), jnp.int32),
            pltpu.VMEM((ROW,), jnp.float32),
            pltpu.VMEM((ROW,), jnp.float32),
            pltpu.VMEM_SHARED((N,), jnp.float32),
            pltpu.VMEM_SHARED((N,), jnp.float32),
        ],
    )(_edge_weights_body)
    return f(src2d, dst2d, as0, as1, ad0, ad1, zn)


def _aggregate_body(src_hbm, dst_hbm, ex0_hbm, ex1_hbm,
                    h00_hbm, h01_hbm, h10_hbm, h11_hbm, znf_hbm,
                    acc_hbm, src_b, dst_b, ex_b, rows_b, six, sg, ssc,
                    acc_sh):
    c = lax.axis_index("c")
    s = lax.axis_index("s")
    base = s * C_ROWS
    tmax = C_ROWS // 3

    def one_pass(h_hbm, ex_hbm, out_idx):
        @pl.when(s == 0)
        def _():
            pltpu.sync_copy(znf_hbm, acc_sh)

        plsc.subcore_barrier()

        def idx_start(j, p):
            pltpu.async_copy(src_hbm.at[base + j], src_b[p], six[p])
            pltpu.async_copy(dst_hbm.at[base + j], dst_b[p], six[p])
            pltpu.async_copy(ex_hbm.at[base + j], ex_b[p], six[p])

        def idx_drain(j, p):
            pltpu.make_async_copy(src_hbm.at[base + j], src_b[p], six[p]).wait()
            pltpu.make_async_copy(dst_hbm.at[base + j], dst_b[p], six[p]).wait()
            pltpu.make_async_copy(ex_hbm.at[base + j], ex_b[p], six[p]).wait()

        def gather_start(p):
            pltpu.async_copy(h_hbm.at[src_b[p]], rows_b[p], sg[p])

        def gather_wait(p):
            pltpu.make_async_copy(h_hbm.at[src_b[p]], rows_b[p], sg[p]).wait()

        def scat_start(p):
            pltpu.make_async_copy(
                rows_b[p], acc_sh.at[dst_b[p]], ssc[p]).start(add=True)

        def scat_wait(p):
            pltpu.make_async_copy(
                rows_b[p], acc_sh.at[dst_b[p]], ssc[p]).wait()

        def scale(p):
            @pl.loop(0, ROW, unroll=8)
            def _(k):
                ek = plsc.load_gather(
                    ex_b[p], [jnp.full((16,), k, jnp.int32)])
                rows_v = rows_b[p]
                for q in range(128 // 16):
                    v = rows_v[k, pl.ds(q * 16, 16)]
                    rows_v[k, pl.ds(q * 16, 16)] = v * ek

        idx_start(0, 0)
        idx_drain(0, 0)
        gather_start(0)
        idx_start(1, 1)

        @pl.loop(0, tmax)
        def _(t):
            j0 = 3 * t
            for i in range(3):
                j = j0 + i
                p, pn1, pn2 = i, (i + 1) % 3, (i + 2) % 3

                def stage_in():
                    idx_drain(j + 1, pn1)
                    gather_start(pn1)

                if i < 2:
                    stage_in()
                else:
                    pl.when(t < tmax - 1)(stage_in)

                gather_wait(p)
                scale(p)
                scat_start(p)

                if i == 0:
                    pl.when(t > 0)(lambda: scat_wait(pn2))
                    idx_start(j + 2, pn2)
                else:
                    scat_wait(pn2)
                    if i == 1:
                        pl.when(t < tmax - 1)(lambda: idx_start(j + 2, pn2))
                    else:
                        pl.when(t < tmax - 1)(lambda: idx_start(j + 2, pn2))

        scat_wait(2)

        plsc.subcore_barrier()

        @pl.when(s == 0)
        def _():
            pltpu.sync_copy(acc_sh, acc_hbm.at[out_idx])

        plsc.subcore_barrier()

    @pl.when(c == 0)
    def _():
        one_pass(h00_hbm, ex0_hbm, 0)
        one_pass(h01_hbm, ex0_hbm, 1)

    @pl.when(c == 1)
    def _():
        one_pass(h10_hbm, ex1_hbm, 2)
        one_pass(h11_hbm, ex1_hbm, 3)


def _aggregate(src2d, dst2d, ex0, ex1, h00, h01, h10, h11, znf):
    mesh = plsc.VectorSubcoreMesh(core_axis_name="c", subcore_axis_name="s",
                                  num_cores=NC, num_subcores=NS)
    f = functools.partial(
        pl.kernel,
        out_type=jax.ShapeDtypeStruct((4, N, 128), jnp.float32),
        mesh=mesh,
        compiler_params=pltpu.CompilerParams(needs_layout_passes=False),
        scratch_types=[
            [pltpu.VMEM((ROW,), jnp.int32)] * 4,
            [pltpu.VMEM((ROW,), jnp.int32)] * 4,
            [pltpu.VMEM((ROW,), jnp.float32)] * 4,
            [pltpu.VMEM((ROW, 128), jnp.float32)] * 4,
            [pltpu.SemaphoreType.DMA] * 4,
            [pltpu.SemaphoreType.DMA] * 4,
            [pltpu.SemaphoreType.DMA] * 4,
            pltpu.VMEM_SHARED((N, 128), jnp.float32),
        ],
    )(_aggregate_body)
    return f(src2d, dst2d, ex0, ex1, h00, h01, h10, h11, znf)


def _finalize_body(acc_ref, denp_ref, bias_ref, gamma_ref, beta_ref, out_ref):
    acc = acc_ref[0]
    dd = denp_ref[...]
    hd = pl.program_id(0) // 2
    denom = jnp.where(hd == 0, dd[0, 0] + dd[1, 0], dd[0, 1] + dd[1, 1])
    denom = denom + 1e-16
    out = acc / denom[:, None] + bias_ref[...][None, :]
    mean = jnp.mean(out, axis=0, keepdims=True)
    var = jnp.mean((out - mean) ** 2, axis=0, keepdims=True)
    out = (out - mean) * jax.lax.rsqrt(var + 1e-5)
    out = out * gamma_ref[...][None, :] + beta_ref[...][None, :]
    out_ref[...] = jnp.maximum(out, 0.0)


def _finalize(acc, denp, bias, gamma, beta):
    vspec = pl.BlockSpec((128,), lambda j: (j,))
    return pl.pallas_call(
        _finalize_body,
        grid=(4,),
        in_specs=[
            pl.BlockSpec((1, N, 128), lambda j: (j, 0, 0)),
            pl.BlockSpec((NC, H, N), lambda j: (0, 0, 0)),
            vspec, vspec, vspec,
        ],
        out_specs=pl.BlockSpec((N, 128), lambda j: (0, j)),
        out_shape=jax.ShapeDtypeStruct((N, H * OUT), jnp.float32),
    )(acc, denp, bias, gamma, beta)


def kernel(x, edge_index, W, att_src, att_dst, bias, bn_gamma, bn_beta):
    loops = jnp.arange(N, dtype=edge_index.dtype)
    pad = jnp.zeros((EP - E_TRUE,), dtype=edge_index.dtype)
    src2d = jnp.concatenate([edge_index[0], loops, pad]).reshape(NROWS, ROW)
    dst2d = jnp.concatenate([edge_index[1], loops, pad]).reshape(NROWS, ROW)
    zn = jnp.zeros((N,), jnp.float32)
    znf = jnp.zeros((N, 128), jnp.float32)

    h00, h01, h10, h11, as0, as1, ad0, ad1 = _project(x, W, att_src, att_dst)
    as0, as1, ad0, ad1 = (a.reshape(N) for a in (as0, as1, ad0, ad1))

    ex0, ex1, denp = _edge_weights(src2d, dst2d, as0, as1, ad0, ad1, zn)
    acc = _aggregate(src2d, dst2d, ex0, ex1, h00, h01, h10, h11, znf)
    return _finalize(acc, denp, bias, bn_gamma, bn_beta)


# speedup vs baseline: 1.5033x; 1.5033x over previous
import functools

import jax
import jax.numpy as jnp
from jax import lax
from jax.experimental import pallas as pl
from jax.experimental.pallas import tpu as pltpu
from jax.experimental.pallas import tpu_sc as plsc

N = 10000
IN = 256
OUT = 256
H = 2
NEG = 0.2
E_RAW = 160000
E_TRUE = E_RAW + N
ROW = 128
EP = 176128
NROWS = EP // ROW
NC = 2
NS = 16
B_ROWS = NROWS // (NC * NS)
C_ROWS = NROWS // NS
NB = 10
BN_ = N // NB


def _proj_body(x_ref, w_ref, asrc_ref, adst_ref,
               h00, h01, h10, h11, as0, as1, ad0, ad1):
    xb = x_ref[...]
    h = jnp.dot(xb, w_ref[...], preferred_element_type=jnp.float32)
    h0 = h[:, :OUT]
    h1 = h[:, OUT:]
    h00[...] = h0[:, :128]
    h01[...] = h0[:, 128:]
    h10[...] = h1[:, :128]
    h11[...] = h1[:, 128:]
    asrc = asrc_ref[...]
    adst = adst_ref[...]
    as0[...] = jnp.sum(h0 * asrc[0][None, :], axis=-1, keepdims=True)
    as1[...] = jnp.sum(h1 * asrc[1][None, :], axis=-1, keepdims=True)
    ad0[...] = jnp.sum(h0 * adst[0][None, :], axis=-1, keepdims=True)
    ad1[...] = jnp.sum(h1 * adst[1][None, :], axis=-1, keepdims=True)


def _project(x, W, att_src, att_dst):
    hspec = pl.BlockSpec((BN_, 128), lambda i: (i, 0))
    aspec = pl.BlockSpec((BN_, 1), lambda i: (i, 0))
    return pl.pallas_call(
        _proj_body,
        grid=(NB,),
        in_specs=[
            pl.BlockSpec((BN_, IN), lambda i: (i, 0)),
            pl.BlockSpec((IN, H * OUT), lambda i: (0, 0)),
            pl.BlockSpec((H, OUT), lambda i: (0, 0)),
            pl.BlockSpec((H, OUT), lambda i: (0, 0)),
        ],
        out_specs=[hspec] * 4 + [aspec] * 4,
        out_shape=[jax.ShapeDtypeStruct((N, 128), jnp.float32)] * 4
        + [jax.ShapeDtypeStruct((N, 1), jnp.float32)] * 4,
    )(x, W, att_src, att_dst)


def _edge_weights_body(src_hbm, dst_hbm, as0_hbm, as1_hbm, ad0_hbm, ad1_hbm,
                       zn_hbm, ex0_hbm, ex1_hbm, denp_hbm,
                       as0_v, as1_v, ad0_v, ad1_v, srcc, dstc, ex0c, ex1c,
                       d0_sh, d1_sh):
    c = lax.axis_index("c")
    s = lax.axis_index("s")
    wid = s * NC + c
    pltpu.sync_copy(as0_hbm, as0_v)
    pltpu.sync_copy(as1_hbm, as1_v)
    pltpu.sync_copy(ad0_hbm, ad0_v)
    pltpu.sync_copy(ad1_hbm, ad1_v)

    @pl.when(s == 0)
    def _():
        pltpu.sync_copy(zn_hbm, d0_sh)
        pltpu.sync_copy(zn_hbm, d1_sh)

    plsc.subcore_barrier()

    base = wid * B_ROWS

    @pl.loop(0, B_ROWS)
    def _(j):
        r = base + j
        pltpu.sync_copy(src_hbm.at[r], srcc)
        pltpu.sync_copy(dst_hbm.at[r], dstc)
        for i in range(ROW // 16):
            s16 = srcc[pl.ds(i * 16, 16)]
            d16 = dstc[pl.ds(i * 16, 16)]
            gid = r * ROW + i * 16 + lax.iota(jnp.int32, 16)
            valid = gid < E_TRUE
            e0 = plsc.load_gather(as0_v, [s16]) + plsc.load_gather(ad0_v, [d16])
            e0 = jnp.where(e0 > 0, e0, NEG * e0)
            x0 = jnp.where(valid, jnp.exp(e0), 0.0)
            ex0c[pl.ds(i * 16, 16)] = x0
            e1 = plsc.load_gather(as1_v, [s16]) + plsc.load_gather(ad1_v, [d16])
            e1 = jnp.where(e1 > 0, e1, NEG * e1)
            x1 = jnp.where(valid, jnp.exp(e1), 0.0)
            ex1c[pl.ds(i * 16, 16)] = x1
        pltpu.sync_copy(ex0c, ex0_hbm.at[r])
        pltpu.sync_copy(ex1c, ex1_hbm.at[r])
        pltpu.sync_copy(ex0c, d0_sh.at[dstc], add=True)
        pltpu.sync_copy(ex1c, d1_sh.at[dstc], add=True)

    plsc.subcore_barrier()

    @pl.when(s == 0)
    def _():
        pltpu.sync_copy(d0_sh, denp_hbm.at[c, 0])
        pltpu.sync_copy(d1_sh, denp_hbm.at[c, 1])


def _edge_weights(src2d, dst2d, as0, as1, ad0, ad1, zn):
    mesh = plsc.VectorSubcoreMesh(core_axis_name="c", subcore_axis_name="s",
                                  num_cores=NC, num_subcores=NS)
    f = functools.partial(
        pl.kernel,
        out_type=[
            jax.ShapeDtypeStruct((NROWS, ROW), jnp.float32),
            jax.ShapeDtypeStruct((NROWS, ROW), jnp.float32),
            jax.ShapeDtypeStruct((NC, H, N), jnp.float32),
        ],
        mesh=mesh,
        compiler_params=pltpu.CompilerParams(needs_layout_passes=False),
        scratch_types=[
            pltpu.VMEM((N,), jnp.float32),
            pltpu.VMEM((N,), jnp.float32),
            pltpu.VMEM((N,), jnp.float32),
            pltpu.VMEM((N,), jnp.float32),
            pltpu.VMEM((ROW,), jnp.int32),
            pltpu.VMEM((ROW,), jnp.int32),
            pltpu.VMEM((ROW,), jnp.float32),
            pltpu.VMEM((ROW,), jnp.float32),
            pltpu.VMEM_SHARED((N,), jnp.float32),
            pltpu.VMEM_SHARED((N,), jnp.float32),
        ],
    )(_edge_weights_body)
    return f(src2d, dst2d, as0, as1, ad0, ad1, zn)


def _aggregate_body(src_hbm, dst_hbm, ex0_hbm, ex1_hbm,
                    h00_hbm, h01_hbm, h10_hbm, h11_hbm, znf_hbm,
                    acc_hbm, src0, src1, dst0, dst1, exc0, exc1,
                    rows0, rows1, six0, six1, sg0, sg1, acc_sh):
    c = lax.axis_index("c")
    s = lax.axis_index("s")
    base = s * C_ROWS

    def one_pass(h_hbm, ex_hbm, out_idx):
        @pl.when(s == 0)
        def _():
            pltpu.sync_copy(znf_hbm, acc_sh)

        plsc.subcore_barrier()

        def idx_start(j, srcc, dstc, exc, sem):
            pltpu.async_copy(src_hbm.at[base + j], srcc, sem)
            pltpu.async_copy(dst_hbm.at[base + j], dstc, sem)
            pltpu.async_copy(ex_hbm.at[base + j], exc, sem)

        def idx_drain(j, srcc, dstc, exc, sem):
            pltpu.make_async_copy(src_hbm.at[base + j], srcc, sem).wait()
            pltpu.make_async_copy(dst_hbm.at[base + j], dstc, sem).wait()
            pltpu.make_async_copy(ex_hbm.at[base + j], exc, sem).wait()

        def scale_scatter(dstc, exc, rows_v):
            @pl.loop(0, ROW, unroll=4)
            def _(k):
                ek = plsc.load_gather(exc, [jnp.full((16,), k, jnp.int32)])
                for q in range(128 // 16):
                    v = rows_v[k, pl.ds(q * 16, 16)]
                    rows_v[k, pl.ds(q * 16, 16)] = v * ek

            pltpu.sync_copy(rows_v, acc_sh.at[dstc], add=True)

        idx_start(0, src0, dst0, exc0, six0)
        idx_drain(0, src0, dst0, exc0, six0)
        pltpu.async_copy(h_hbm.at[src0], rows0, sg0)
        idx_start(1, src1, dst1, exc1, six1)

        @pl.loop(0, C_ROWS // 2)
        def _(t):
            j0 = 2 * t
            idx_drain(j0 + 1, src1, dst1, exc1, six1)
            pltpu.async_copy(h_hbm.at[src1], rows1, sg1)
            pltpu.make_async_copy(h_hbm.at[src0], rows0, sg0).wait()
            scale_scatter(dst0, exc0, rows0)

            @pl.when(j0 + 2 < C_ROWS)
            def _():
                idx_start(j0 + 2, src0, dst0, exc0, six0)
                idx_drain(j0 + 2, src0, dst0, exc0, six0)
                pltpu.async_copy(h_hbm.at[src0], rows0, sg0)

            pltpu.make_async_copy(h_hbm.at[src1], rows1, sg1).wait()
            scale_scatter(dst1, exc1, rows1)

            @pl.when(j0 + 3 < C_ROWS)
            def _():
                idx_start(j0 + 3, src1, dst1, exc1, six1)

        plsc.subcore_barrier()

        @pl.when(s == 0)
        def _():
            pltpu.sync_copy(acc_sh, acc_hbm.at[out_idx])

        plsc.subcore_barrier()

    @pl.when(c == 0)
    def _():
        one_pass(h00_hbm, ex0_hbm, 0)
        one_pass(h01_hbm, ex0_hbm, 1)

    @pl.when(c == 1)
    def _():
        one_pass(h10_hbm, ex1_hbm, 2)
        one_pass(h11_hbm, ex1_hbm, 3)


def _aggregate(src2d, dst2d, ex0, ex1, h00, h01, h10, h11, znf):
    mesh = plsc.VectorSubcoreMesh(core_axis_name="c", subcore_axis_name="s",
                                  num_cores=NC, num_subcores=NS)
    f = functools.partial(
        pl.kernel,
        out_type=jax.ShapeDtypeStruct((4, N, 128), jnp.float32),
        mesh=mesh,
        compiler_params=pltpu.CompilerParams(needs_layout_passes=False),
        scratch_types=[
            pltpu.VMEM((ROW,), jnp.int32),
            pltpu.VMEM((ROW,), jnp.int32),
            pltpu.VMEM((ROW,), jnp.int32),
            pltpu.VMEM((ROW,), jnp.int32),
            pltpu.VMEM((ROW,), jnp.float32),
            pltpu.VMEM((ROW,), jnp.float32),
            pltpu.VMEM((ROW, 128), jnp.float32),
            pltpu.VMEM((ROW, 128), jnp.float32),
            pltpu.SemaphoreType.DMA,
            pltpu.SemaphoreType.DMA,
            pltpu.SemaphoreType.DMA,
            pltpu.SemaphoreType.DMA,
            pltpu.VMEM_SHARED((N, 128), jnp.float32),
        ],
    )(_aggregate_body)
    return f(src2d, dst2d, ex0, ex1, h00, h01, h10, h11, znf)


def _finalize_body(acc_ref, denp_ref, bias_ref, gamma_ref, beta_ref, out_ref):
    acc = acc_ref[0]
    dd = denp_ref[...]
    hd = pl.program_id(0) // 2
    denom = jnp.where(hd == 0, dd[0, 0] + dd[1, 0], dd[0, 1] + dd[1, 1])
    denom = denom + 1e-16
    out = acc / denom[:, None] + bias_ref[...][None, :]
    mean = jnp.mean(out, axis=0, keepdims=True)
    var = jnp.mean((out - mean) ** 2, axis=0, keepdims=True)
    out = (out - mean) * jax.lax.rsqrt(var + 1e-5)
    out = out * gamma_ref[...][None, :] + beta_ref[...][None, :]
    out_ref[...] = jnp.maximum(out, 0.0)


def _finalize(acc, denp, bias, gamma, beta):
    vspec = pl.BlockSpec((128,), lambda j: (j,))
    return pl.pallas_call(
        _finalize_body,
        grid=(4,),
        in_specs=[
            pl.BlockSpec((1, N, 128), lambda j: (j, 0, 0)),
            pl.BlockSpec((NC, H, N), lambda j: (0, 0, 0)),
            vspec, vspec, vspec,
        ],
        out_specs=pl.BlockSpec((N, 128), lambda j: (0, j)),
        out_shape=jax.ShapeDtypeStruct((N, H * OUT), jnp.float32),
    )(acc, denp, bias, gamma, beta)


def kernel(x, edge_index, W, att_src, att_dst, bias, bn_gamma, bn_beta):
    loops = jnp.arange(N, dtype=edge_index.dtype)
    pad = jnp.zeros((EP - E_TRUE,), dtype=edge_index.dtype)
    src2d = jnp.concatenate([edge_index[0], loops, pad]).reshape(NROWS, ROW)
    dst2d = jnp.concatenate([edge_index[1], loops, pad]).reshape(NROWS, ROW)
    zn = jnp.zeros((N,), jnp.float32)
    znf = jnp.zeros((N, 128), jnp.float32)

    h00, h01, h10, h11, as0, as1, ad0, ad1 = _project(x, W, att_src, att_dst)
    as0, as1, ad0, ad1 = (a.reshape(N) for a in (as0, as1, ad0, ad1))

    ex0, ex1, denp = _edge_weights(src2d, dst2d, as0, as1, ad0, ad1, zn)
    acc = _aggregate(src2d, dst2d, ex0, ex1, h00, h01, h10, h11, znf)
    return _finalize(acc, denp, bias, bn_gamma, bn_beta)


# speedup vs baseline: 1.5407x; 1.0249x over previous
import functools

import jax
import jax.numpy as jnp
from jax import lax
from jax.experimental import pallas as pl
from jax.experimental.pallas import tpu as pltpu
from jax.experimental.pallas import tpu_sc as plsc

N = 10000
IN = 256
OUT = 256
H = 2
NEG = 0.2
E_RAW = 160000
E_TRUE = E_RAW + N
ROW = 128
EP = 176128
NROWS = EP // ROW
NC = 2
NS = 16
B_ROWS = NROWS // (NC * NS)
C_ROWS = NROWS // NS
NB = 10
BN_ = N // NB


def _proj_body(x_ref, w_ref, asrc_ref, adst_ref,
               h00, h01, h10, h11, as0, as1, ad0, ad1):
    xb = x_ref[...]
    h = jnp.dot(xb, w_ref[...], preferred_element_type=jnp.float32)
    h0 = h[:, :OUT]
    h1 = h[:, OUT:]
    h00[...] = h0[:, :128]
    h01[...] = h0[:, 128:]
    h10[...] = h1[:, :128]
    h11[...] = h1[:, 128:]
    asrc = asrc_ref[...]
    adst = adst_ref[...]
    as0[...] = jnp.sum(h0 * asrc[0][None, :], axis=-1, keepdims=True)
    as1[...] = jnp.sum(h1 * asrc[1][None, :], axis=-1, keepdims=True)
    ad0[...] = jnp.sum(h0 * adst[0][None, :], axis=-1, keepdims=True)
    ad1[...] = jnp.sum(h1 * adst[1][None, :], axis=-1, keepdims=True)


def _project(x, W, att_src, att_dst):
    hspec = pl.BlockSpec((BN_, 128), lambda i: (i, 0))
    aspec = pl.BlockSpec((BN_, 1), lambda i: (i, 0))
    return pl.pallas_call(
        _proj_body,
        grid=(NB,),
        in_specs=[
            pl.BlockSpec((BN_, IN), lambda i: (i, 0)),
            pl.BlockSpec((IN, H * OUT), lambda i: (0, 0)),
            pl.BlockSpec((H, OUT), lambda i: (0, 0)),
            pl.BlockSpec((H, OUT), lambda i: (0, 0)),
        ],
        out_specs=[hspec] * 4 + [aspec] * 4,
        out_shape=[jax.ShapeDtypeStruct((N, 128), jnp.float32)] * 4
        + [jax.ShapeDtypeStruct((N, 1), jnp.float32)] * 4,
    )(x, W, att_src, att_dst)


def _edge_weights_body(src_hbm, dst_hbm, as0_hbm, as1_hbm, ad0_hbm, ad1_hbm,
                       zn_hbm, ex0_hbm, ex1_hbm, denp_hbm,
                       as0_v, as1_v, ad0_v, ad1_v, srcc, dstc, ex0c, ex1c,
                       d0_sh, d1_sh):
    c = lax.axis_index("c")
    s = lax.axis_index("s")
    wid = s * NC + c
    pltpu.sync_copy(as0_hbm, as0_v)
    pltpu.sync_copy(as1_hbm, as1_v)
    pltpu.sync_copy(ad0_hbm, ad0_v)
    pltpu.sync_copy(ad1_hbm, ad1_v)

    @pl.when(s == 0)
    def _():
        pltpu.sync_copy(zn_hbm, d0_sh)
        pltpu.sync_copy(zn_hbm, d1_sh)

    plsc.subcore_barrier()

    base = wid * B_ROWS

    @pl.loop(0, B_ROWS)
    def _(j):
        r = base + j
        pltpu.sync_copy(src_hbm.at[r], srcc)
        pltpu.sync_copy(dst_hbm.at[r], dstc)
        for i in range(ROW // 16):
            s16 = srcc[pl.ds(i * 16, 16)]
            d16 = dstc[pl.ds(i * 16, 16)]
            gid = r * ROW + i * 16 + lax.iota(jnp.int32, 16)
            valid = gid < E_TRUE
            e0 = plsc.load_gather(as0_v, [s16]) + plsc.load_gather(ad0_v, [d16])
            e0 = jnp.where(e0 > 0, e0, NEG * e0)
            x0 = jnp.where(valid, jnp.exp(e0), 0.0)
            ex0c[pl.ds(i * 16, 16)] = x0
            e1 = plsc.load_gather(as1_v, [s16]) + plsc.load_gather(ad1_v, [d16])
            e1 = jnp.where(e1 > 0, e1, NEG * e1)
            x1 = jnp.where(valid, jnp.exp(e1), 0.0)
            ex1c[pl.ds(i * 16, 16)] = x1
        pltpu.sync_copy(ex0c, ex0_hbm.at[r])
        pltpu.sync_copy(ex1c, ex1_hbm.at[r])
        pltpu.sync_copy(ex0c, d0_sh.at[dstc], add=True)
        pltpu.sync_copy(ex1c, d1_sh.at[dstc], add=True)

    plsc.subcore_barrier()

    @pl.when(s == 0)
    def _():
        pltpu.sync_copy(d0_sh, denp_hbm.at[c, 0])
        pltpu.sync_copy(d1_sh, denp_hbm.at[c, 1])


def _edge_weights(src2d, dst2d, as0, as1, ad0, ad1, zn):
    mesh = plsc.VectorSubcoreMesh(core_axis_name="c", subcore_axis_name="s",
                                  num_cores=NC, num_subcores=NS)
    f = functools.partial(
        pl.kernel,
        out_type=[
            jax.ShapeDtypeStruct((NROWS, ROW), jnp.float32),
            jax.ShapeDtypeStruct((NROWS, ROW), jnp.float32),
            jax.ShapeDtypeStruct((NC, H, N), jnp.float32),
        ],
        mesh=mesh,
        compiler_params=pltpu.CompilerParams(needs_layout_passes=False),
        scratch_types=[
            pltpu.VMEM((N,), jnp.float32),
            pltpu.VMEM((N,), jnp.float32),
            pltpu.VMEM((N,), jnp.float32),
            pltpu.VMEM((N,), jnp.float32),
            pltpu.VMEM((ROW,), jnp.int32),
            pltpu.VMEM((ROW,), jnp.int32),
            pltpu.VMEM((ROW,), jnp.float32),
            pltpu.VMEM((ROW,), jnp.float32),
            pltpu.VMEM_SHARED((N,), jnp.float32),
            pltpu.VMEM_SHARED((N,), jnp.float32),
        ],
    )(_edge_weights_body)
    return f(src2d, dst2d, as0, as1, ad0, ad1, zn)


def _aggregate_body(src_hbm, dst_hbm, ex0_hbm, ex1_hbm,
                    h00_hbm, h01_hbm, h10_hbm, h11_hbm, znf_hbm,
                    acc_hbm, src0, src1, dst0, dst1, exc0, exc1,
                    rows0, rows1, six0, six1, sg0, sg1, acc_sh):
    c = lax.axis_index("c")
    s = lax.axis_index("s")
    base = s * C_ROWS

    def one_pass(h_hbm, ex_hbm, out_idx):
        @pl.when(s == 0)
        def _():
            pltpu.sync_copy(znf_hbm, acc_sh)

        plsc.subcore_barrier()

        def idx_start(j, srcc, dstc, exc, sem):
            pltpu.async_copy(src_hbm.at[base + j], srcc, sem)
            pltpu.async_copy(dst_hbm.at[base + j], dstc, sem)
            pltpu.async_copy(ex_hbm.at[base + j], exc, sem)

        def idx_drain(j, srcc, dstc, exc, sem):
            pltpu.make_async_copy(src_hbm.at[base + j], srcc, sem).wait()
            pltpu.make_async_copy(dst_hbm.at[base + j], dstc, sem).wait()
            pltpu.make_async_copy(ex_hbm.at[base + j], exc, sem).wait()

        def scale_scatter(dstc, exc, rows_v):
            @pl.loop(0, ROW, unroll=8)
            def _(k):
                ek = plsc.load_gather(exc, [jnp.full((16,), k, jnp.int32)])
                for q in range(128 // 16):
                    v = rows_v[k, pl.ds(q * 16, 16)]
                    rows_v[k, pl.ds(q * 16, 16)] = v * ek

            pltpu.sync_copy(rows_v, acc_sh.at[dstc], add=True)

        idx_start(0, src0, dst0, exc0, six0)
        idx_drain(0, src0, dst0, exc0, six0)
        pltpu.async_copy(h_hbm.at[src0], rows0, sg0)
        idx_start(1, src1, dst1, exc1, six1)

        @pl.loop(0, C_ROWS // 2)
        def _(t):
            j0 = 2 * t
            idx_drain(j0 + 1, src1, dst1, exc1, six1)
            pltpu.async_copy(h_hbm.at[src1], rows1, sg1)
            pltpu.make_async_copy(h_hbm.at[src0], rows0, sg0).wait()
            scale_scatter(dst0, exc0, rows0)

            @pl.when(j0 + 2 < C_ROWS)
            def _():
                idx_start(j0 + 2, src0, dst0, exc0, six0)
                idx_drain(j0 + 2, src0, dst0, exc0, six0)
                pltpu.async_copy(h_hbm.at[src0], rows0, sg0)

            pltpu.make_async_copy(h_hbm.at[src1], rows1, sg1).wait()
            scale_scatter(dst1, exc1, rows1)

            @pl.when(j0 + 3 < C_ROWS)
            def _():
                idx_start(j0 + 3, src1, dst1, exc1, six1)

        plsc.subcore_barrier()

        @pl.when(s == 0)
        def _():
            pltpu.sync_copy(acc_sh, acc_hbm.at[out_idx])

        plsc.subcore_barrier()

    @pl.when(c == 0)
    def _():
        one_pass(h00_hbm, ex0_hbm, 0)
        one_pass(h01_hbm, ex0_hbm, 1)

    @pl.when(c == 1)
    def _():
        one_pass(h10_hbm, ex1_hbm, 2)
        one_pass(h11_hbm, ex1_hbm, 3)


def _aggregate(src2d, dst2d, ex0, ex1, h00, h01, h10, h11, znf):
    mesh = plsc.VectorSubcoreMesh(core_axis_name="c", subcore_axis_name="s",
                                  num_cores=NC, num_subcores=NS)
    f = functools.partial(
        pl.kernel,
        out_type=jax.ShapeDtypeStruct((4, N, 128), jnp.float32),
        mesh=mesh,
        compiler_params=pltpu.CompilerParams(needs_layout_passes=False),
        scratch_types=[
            pltpu.VMEM((ROW,), jnp.int32),
            pltpu.VMEM((ROW,), jnp.int32),
            pltpu.VMEM((ROW,), jnp.int32),
            pltpu.VMEM((ROW,), jnp.int32),
            pltpu.VMEM((ROW,), jnp.float32),
            pltpu.VMEM((ROW,), jnp.float32),
            pltpu.VMEM((ROW, 128), jnp.float32),
            pltpu.VMEM((ROW, 128), jnp.float32),
            pltpu.SemaphoreType.DMA,
            pltpu.SemaphoreType.DMA,
            pltpu.SemaphoreType.DMA,
            pltpu.SemaphoreType.DMA,
            pltpu.VMEM_SHARED((N, 128), jnp.float32),
        ],
    )(_aggregate_body)
    return f(src2d, dst2d, ex0, ex1, h00, h01, h10, h11, znf)


def _finalize_body(acc_ref, denp_ref, bias_ref, gamma_ref, beta_ref, out_ref):
    acc = acc_ref[0]
    dd = denp_ref[...]
    hd = pl.program_id(0) // 2
    denom = jnp.where(hd == 0, dd[0, 0] + dd[1, 0], dd[0, 1] + dd[1, 1])
    denom = denom + 1e-16
    out = acc / denom[:, None] + bias_ref[...][None, :]
    mean = jnp.mean(out, axis=0, keepdims=True)
    var = jnp.mean((out - mean) ** 2, axis=0, keepdims=True)
    out = (out - mean) * jax.lax.rsqrt(var + 1e-5)
    out = out * gamma_ref[...][None, :] + beta_ref[...][None, :]
    out_ref[...] = jnp.maximum(out, 0.0)


def _finalize(acc, denp, bias, gamma, beta):
    vspec = pl.BlockSpec((128,), lambda j: (j,))
    return pl.pallas_call(
        _finalize_body,
        grid=(4,),
        in_specs=[
            pl.BlockSpec((1, N, 128), lambda j: (j, 0, 0)),
            pl.BlockSpec((NC, H, N), lambda j: (0, 0, 0)),
            vspec, vspec, vspec,
        ],
        out_specs=pl.BlockSpec((N, 128), lambda j: (0, j)),
        out_shape=jax.ShapeDtypeStruct((N, H * OUT), jnp.float32),
    )(acc, denp, bias, gamma, beta)


def kernel(x, edge_index, W, att_src, att_dst, bias, bn_gamma, bn_beta):
    loops = jnp.arange(N, dtype=edge_index.dtype)
    pad = jnp.zeros((EP - E_TRUE,), dtype=edge_index.dtype)
    src2d = jnp.concatenate([edge_index[0], loops, pad]).reshape(NROWS, ROW)
    dst2d = jnp.concatenate([edge_index[1], loops, pad]).reshape(NROWS, ROW)
    zn = jnp.zeros((N,), jnp.float32)
    znf = jnp.zeros((N, 128), jnp.float32)

    h00, h01, h10, h11, as0, as1, ad0, ad1 = _project(x, W, att_src, att_dst)
    as0, as1, ad0, ad1 = (a.reshape(N) for a in (as0, as1, ad0, ad1))

    ex0, ex1, denp = _edge_weights(src2d, dst2d, as0, as1, ad0, ad1, zn)
    acc = _aggregate(src2d, dst2d, ex0, ex1, h00, h01, h10, h11, znf)
    return _finalize(acc, denp, bias, bn_gamma, bn_beta)
